# Initial kernel scaffold; baseline (speedup 1.0000x reference)
import jax
import jax.numpy as jnp
from jax.experimental import pallas as pl


def kernel(user_ids, item_ids, user_embedding, item_embedding, W_user, b_user, W_item, b_item):
    raise NotImplementedError("write your pallas kernel here")


# speedup vs baseline: 3.2174x; 3.2174x over previous
import functools

import jax
import jax.numpy as jnp
from jax import lax
from jax.experimental import pallas as pl
from jax.experimental.pallas import tpu as pltpu
from jax.experimental.pallas import tpu_sc as plsc

D = 128
NC, NS = 2, 16
NW = NC * NS
CHUNK = 128


def _sc_gather(uids2, iids2, utab, itab):
    n_chunks = uids2.shape[0]
    B = n_chunks * CHUNK
    kpw = n_chunks // NW
    mesh = plsc.VectorSubcoreMesh(
        core_axis_name="c", subcore_axis_name="s",
        num_cores=NC, num_subcores=NS)

    @functools.partial(
        pl.kernel,
        out_type=(jax.ShapeDtypeStruct((B, D), jnp.float32),
                  jax.ShapeDtypeStruct((B, D), jnp.float32)),
        mesh=mesh,
        scratch_types=[
            pltpu.VMEM((kpw, CHUNK), jnp.int32),
            pltpu.VMEM((kpw, CHUNK), jnp.int32),
            pltpu.VMEM((CHUNK, D), jnp.float32),
            pltpu.VMEM((CHUNK, D), jnp.float32),
            pltpu.SemaphoreType.DMA,
            pltpu.SemaphoreType.DMA,
        ],
    )
    def k(uid_hbm, iid_hbm, utab_hbm, itab_hbm, uout_hbm, iout_hbm,
          idx_u, idx_i, buf0, buf1, sem0, sem1):
        wid = lax.axis_index("s") * NC + lax.axis_index("c")
        cbase = wid * kpw
        pltpu.sync_copy(uid_hbm.at[pl.ds(cbase, kpw)], idx_u)
        pltpu.sync_copy(iid_hbm.at[pl.ds(cbase, kpw)], idx_i)

        bufs = [buf0, buf1]
        sems = [sem0, sem1]
        total = 2 * kpw

        def chunk_info(c):
            if c < kpw:
                return utab_hbm, idx_u, c, uout_hbm
            return itab_hbm, idx_i, c - kpw, iout_hbm

        def writeback(c):
            _, _, j, out = chunk_info(c)
            row0 = (cbase + j) * CHUNK
            pltpu.sync_copy(bufs[c % 2], out.at[pl.ds(row0, CHUNK)])

        copies = [None, None]
        for c in range(total):
            b = c % 2
            if copies[b] is not None:
                copies[b].wait()
                writeback(c - 2)
            tab, idx, j, _ = chunk_info(c)
            copies[b] = pltpu.async_copy(tab.at[idx.at[j]], bufs[b], sems[b])
        for c in (total - 2, total - 1):
            copies[c % 2].wait()
            writeback(c)

    return k(uids2, iids2, utab, itab)


def _tc_project(xu, xi, Wu, bu, Wi, bi):
    B = xu.shape[0]
    BM = 2048
    dn = (((1,), (1,)), ((), ()))

    def body(xu_ref, xi_ref, wu_ref, bu_ref, wi_ref, bi_ref, ou_ref, oi_ref):
        ou_ref[...] = lax.dot_general(
            xu_ref[...], wu_ref[...], dn,
            preferred_element_type=jnp.float32) + bu_ref[...]
        oi_ref[...] = lax.dot_general(
            xi_ref[...], wi_ref[...], dn,
            preferred_element_type=jnp.float32) + bi_ref[...]

    return pl.pallas_call(
        body,
        grid=(B // BM,),
        in_specs=[
            pl.BlockSpec((BM, D), lambda i: (i, 0)),
            pl.BlockSpec((BM, D), lambda i: (i, 0)),
            pl.BlockSpec((D, D), lambda i: (0, 0)),
            pl.BlockSpec((1, D), lambda i: (0, 0)),
            pl.BlockSpec((D, D), lambda i: (0, 0)),
            pl.BlockSpec((1, D), lambda i: (0, 0)),
        ],
        out_specs=[
            pl.BlockSpec((BM, D), lambda i: (i, 0)),
            pl.BlockSpec((BM, D), lambda i: (i, 0)),
        ],
        out_shape=[
            jax.ShapeDtypeStruct((B, D), jnp.float32),
            jax.ShapeDtypeStruct((B, D), jnp.float32),
        ],
    )(xu, xi, Wu, bu.reshape(1, D), Wi, bi.reshape(1, D))


def kernel(user_ids, item_ids, user_embedding, item_embedding,
           W_user, b_user, W_item, b_item):
    B = user_ids.shape[0]
    uids2 = user_ids.astype(jnp.int32).reshape(B // CHUNK, CHUNK)
    iids2 = item_ids.astype(jnp.int32).reshape(B // CHUNK, CHUNK)
    gu, gi = _sc_gather(uids2, iids2, user_embedding, item_embedding)
    ou, oi = _tc_project(gu, gi, W_user, b_user, W_item, b_item)
    return (ou, oi)
